# Initial kernel scaffold; baseline (speedup 1.0000x reference)
import jax
import jax.numpy as jnp
from jax.experimental import pallas as pl


def kernel(X0, X1, X2, L0_rows, L0_cols, L0_vals, L1_rows, L1_cols, L1_vals, L2_rows, L2_cols, L2_vals, B2D3_rows, B2D3_cols, B2D3_vals, D2B1TD1inv_rows, D2B1TD1inv_cols, D2B1TD1inv_vals, D1invB1_rows, D1invB1_cols, D1invB1_vals, B2TD2inv_rows, B2TD2inv_cols, B2TD2inv_vals, batch0, batch1, batch2, W1, b1, W2, b2, W3, b3, Wout, bout):
    raise NotImplementedError("write your pallas kernel here")


# speedup vs baseline: 2.4160x; 2.4160x over previous
import functools

import jax
import jax.numpy as jnp
from jax import lax
from jax.experimental import pallas as pl
from jax.experimental.pallas import tpu as pltpu
from jax.experimental.pallas import tpu_sc as plsc

F32 = jnp.float32
I32 = jnp.int32

NC = 2
NS = 16
C = 128
WC = 200

_mesh = plsc.VectorSubcoreMesh(core_axis_name="c", subcore_axis_name="s")


def _cdiv(a, b):
    return -(-a // b)


def _spmm_acc(n_out, alpha, edge_counts):
    Nh = n_out // 2
    ACC = _cdiv(Nh + 1, C) * C
    nops = len(edge_counts)

    def body(*refs):
        ins = refs[: 4 * nops]
        out_h = refs[4 * nops]
        colv, idxv, valv, gbuf, obuf, acc, sem = refs[4 * nops + 1:]
        cid = lax.axis_index("c")
        sid = lax.axis_index("s")

        zero16 = jnp.zeros((16,), F32)

        @pl.loop(0, C)
        def _zg(r):
            gbuf[r, pl.ds(0, 16)] = zero16
            gbuf[r, pl.ds(16, 16)] = zero16

        n_zc = ACC // C

        @pl.loop(0, _cdiv(n_zc, NS))
        def _za(j):
            z = sid + j * NS

            @pl.when(z < n_zc)
            def _():
                pltpu.sync_copy(gbuf, acc.at[pl.ds(z * C, C)])

        plsc.subcore_barrier()

        lane = lax.iota(I32, 16)
        row_lo = cid * Nh

        for oi in range(nops):
            rows_h, cols_h, vals_h, y_h = ins[4 * oi: 4 * oi + 4]
            E = edge_counts[oi]
            n_ch = _cdiv(E, C)

            @pl.loop(0, _cdiv(n_ch, NS))
            def _edges(j, rows_h=rows_h, cols_h=cols_h, vals_h=vals_h,
                       y_h=y_h, E=E, n_ch=n_ch):
                ch = sid + j * NS

                @pl.when(ch < n_ch)
                def _():
                    base0 = ch * C
                    base = jnp.minimum(base0, E - C)
                    lane_lo = base0 - base
                    pltpu.sync_copy(cols_h.at[pl.ds(base, C)], colv)
                    pltpu.sync_copy(rows_h.at[pl.ds(base, C)], idxv)
                    pltpu.sync_copy(vals_h.at[pl.ds(base, C)], valv)
                    pltpu.async_copy(y_h.at[colv], gbuf, sem).wait()
                    for g in range(C // 16):
                        dv = idxv[pl.ds(g * 16, 16)]
                        lv = dv - row_lo
                        ok = (lv >= 0) & (lv < Nh) & ((lane + g * 16) >= lane_lo)
                        idxv[pl.ds(g * 16, 16)] = jnp.where(ok, lv, Nh)
                        for e in range(16):
                            r = g * 16 + e
                            sv = plsc.load_gather(
                                valv, [jnp.full((16,), r, I32)])
                            gbuf[r, pl.ds(0, 16)] = gbuf[r, pl.ds(0, 16)] * sv
                            gbuf[r, pl.ds(16, 16)] = gbuf[r, pl.ds(16, 16)] * sv
                    pltpu.sync_copy(gbuf, acc.at[idxv], add=True)

        plsc.subcore_barrier()

        n_wc = Nh // WC
        a = jnp.float32(alpha)

        @pl.loop(0, _cdiv(n_wc, NS))
        def _wb(j):
            w = sid + j * NS

            @pl.when(w < n_wc)
            def _():
                pltpu.sync_copy(acc.at[pl.ds(w * WC, WC)], obuf)

                @pl.loop(0, WC)
                def _r(r):
                    lo = obuf[r, pl.ds(0, 16)]
                    obuf[r, pl.ds(0, 16)] = jnp.maximum(lo, 0.0) * a
                    hi = obuf[r, pl.ds(16, 16)]
                    obuf[r, pl.ds(16, 16)] = jnp.maximum(hi, 0.0) * a

                pltpu.sync_copy(obuf, out_h.at[pl.ds(row_lo + w * WC, WC)])

    return pl.kernel(
        body,
        out_type=jax.ShapeDtypeStruct((n_out, 32), F32),
        mesh=_mesh,
        compiler_params=pltpu.CompilerParams(needs_layout_passes=False, use_tc_tiling_on_sc=False),
        scratch_types=[
            pltpu.VMEM((C,), I32),
            pltpu.VMEM((C,), I32),
            pltpu.VMEM((C,), F32),
            pltpu.VMEM((C, 32), F32),
            pltpu.VMEM((WC, 32), F32),
            pltpu.VMEM_SHARED((ACC, 32), F32),
            pltpu.SemaphoreType.DMA,
        ],
    )


def _pool(n_rows):
    ACC = 128

    def body(x_h, b_h, sums_h, cnts_h, idxv, gbuf, onesb, obuf, acc_s, acc_c,
             sem):
        del sem
        cid = lax.axis_index("c")
        sid = lax.axis_index("s")
        wid = sid * NC + cid

        zero16 = jnp.zeros((16,), F32)
        one16 = jnp.ones((16,), F32)

        @pl.loop(0, C)
        def _init(r):
            gbuf[r, pl.ds(0, 16)] = zero16
            gbuf[r, pl.ds(16, 16)] = zero16
            onesb[r, pl.ds(0, 16)] = one16
            onesb[r, pl.ds(16, 16)] = one16

        @pl.when(sid == 0)
        def _():
            pltpu.sync_copy(gbuf, acc_s)
            pltpu.sync_copy(gbuf, acc_c)

        plsc.subcore_barrier()

        lane = lax.iota(I32, 16)
        n_ch = _cdiv(n_rows, C)

        @pl.loop(0, _cdiv(n_ch, NC * NS))
        def _rows(j):
            ch = wid + j * NC * NS

            @pl.when(ch < n_ch)
            def _():
                base0 = ch * C
                base = jnp.minimum(base0, n_rows - C)
                lane_lo = base0 - base
                pltpu.sync_copy(x_h.at[pl.ds(base, C)], gbuf)
                pltpu.sync_copy(b_h.at[pl.ds(base, C)], idxv)
                for g in range(C // 16):
                    bv = idxv[pl.ds(g * 16, 16)]
                    ok = (bv >= 0) & (bv < 64) & ((lane + g * 16) >= lane_lo)
                    idxv[pl.ds(g * 16, 16)] = jnp.where(ok, bv, 64)
                pltpu.sync_copy(gbuf, acc_s.at[idxv], add=True)
                pltpu.sync_copy(onesb, acc_c.at[idxv], add=True)

        plsc.subcore_barrier()

        @pl.when(sid == 0)
        def _():
            pltpu.sync_copy(acc_s.at[pl.ds(0, 64)], obuf)
            pltpu.sync_copy(obuf, sums_h.at[cid])
            pltpu.sync_copy(acc_c.at[pl.ds(0, 64)], obuf)
            pltpu.sync_copy(obuf, cnts_h.at[cid])

    return pl.kernel(
        body,
        out_type=(jax.ShapeDtypeStruct((NC, 64, 32), F32),
                  jax.ShapeDtypeStruct((NC, 64, 32), F32)),
        mesh=_mesh,
        compiler_params=pltpu.CompilerParams(
            needs_layout_passes=False, use_tc_tiling_on_sc=False),
        scratch_types=[
            pltpu.VMEM((C,), I32),
            pltpu.VMEM((C, 32), F32),
            pltpu.VMEM((C, 32), F32),
            pltpu.VMEM((64, 32), F32),
            pltpu.VMEM_SHARED((ACC, 32), F32),
            pltpu.VMEM_SHARED((ACC, 32), F32),
            pltpu.SemaphoreType.DMA,
        ],
    )


def _dense(X, Ws, bs, block_rows=1000):
    n, k = X.shape
    ny = len(Ws)
    W = jnp.stack(Ws)
    b = jnp.stack(bs).reshape(ny, 1, 32)

    def body(x_ref, w_ref, b_ref, *outs):
        x = x_ref[...]
        for i in range(ny):
            outs[i][...] = (
                jnp.dot(x, w_ref[i], preferred_element_type=F32) + b_ref[i]
            )

    return pl.pallas_call(
        body,
        grid=(n // block_rows,),
        in_specs=[
            pl.BlockSpec((block_rows, k), lambda i: (i, 0)),
            pl.BlockSpec((ny, k, 32), lambda i: (0, 0, 0)),
            pl.BlockSpec((ny, 1, 32), lambda i: (0, 0, 0)),
        ],
        out_specs=[pl.BlockSpec((block_rows, 32), lambda i: (i, 0))] * ny,
        out_shape=[jax.ShapeDtypeStruct((n, 32), F32)] * ny,
    )(X, W, b)


def _final(s0, c0, s1, c1, s2, c2, Wout, bout):
    def body(s0r, c0r, s1r, c1r, s2r, c2r, wr, br, outr):
        def pooled(sr, cr):
            return (sr[0] + sr[1]) / jnp.maximum(cr[0] + cr[1], 1.0)

        cat = jnp.concatenate(
            [pooled(s0r, c0r), pooled(s1r, c1r), pooled(s2r, c2r)], axis=1)
        z = jnp.dot(cat, wr[...], preferred_element_type=F32) + br[...]
        z = z - jnp.max(z, axis=1, keepdims=True)
        ez = jnp.exp(z)
        outr[...] = ez / jnp.sum(ez, axis=1, keepdims=True)

    return pl.pallas_call(
        body,
        out_shape=jax.ShapeDtypeStruct((64, 32), F32),
    )(s0, c0, s1, c1, s2, c2, Wout, bout.reshape(1, 32))


def _pad_w(W, b):
    kin, kout = W.shape[1], W.shape[2]
    kin_pad = 128 if kin == 128 else 32
    Wp = jnp.zeros((7, kin_pad, 32), F32).at[:, :kin, :kout].set(W)
    bp = jnp.zeros((7, 32), F32).at[:, :kout].set(b)
    return Wp, bp


def kernel(X0, X1, X2,
           L0_rows, L0_cols, L0_vals,
           L1_rows, L1_cols, L1_vals,
           L2_rows, L2_cols, L2_vals,
           B2D3_rows, B2D3_cols, B2D3_vals,
           D2B1TD1inv_rows, D2B1TD1inv_cols, D2B1TD1inv_vals,
           D1invB1_rows, D1invB1_cols, D1invB1_vals,
           B2TD2inv_rows, B2TD2inv_cols, B2TD2inv_vals,
           batch0, batch1, batch2,
           W1, b1, W2, b2, W3, b3, Wout, bout):
    ii = lambda x: x.astype(I32)
    ff = lambda x: x.astype(F32)

    ops = {
        "L0": (ii(L0_rows), ii(L0_cols), ff(L0_vals)),
        "L1": (ii(L1_rows), ii(L1_cols), ff(L1_vals)),
        "L2": (ii(L2_rows), ii(L2_cols), ff(L2_vals)),
        "B2D3": (ii(B2D3_rows), ii(B2D3_cols), ff(B2D3_vals)),
        "D2B1TD1inv": (ii(D2B1TD1inv_rows), ii(D2B1TD1inv_cols),
                       ff(D2B1TD1inv_vals)),
        "D1invB1": (ii(D1invB1_rows), ii(D1invB1_cols), ff(D1invB1_vals)),
        "B2TD2inv": (ii(B2TD2inv_rows), ii(B2TD2inv_cols), ff(B2TD2inv_vals)),
    }

    def layer(x0, x1, x2, W, b):
        Wp, bp = _pad_w(W, b)
        y_n2n, y_n2e = _dense(x0, [Wp[0], Wp[1]], [bp[0], bp[1]])
        y_e2e, y_e2n, y_e2t = _dense(x1, [Wp[2], Wp[3], Wp[4]],
                                     [bp[2], bp[3], bp[4]])
        y_t2e, y_t2t = _dense(x2, [Wp[5], Wp[6]], [bp[5], bp[6]])

        def run(n_out, alpha, pairs):
            counts = tuple(p[0][0].shape[0] for p in pairs)
            args = []
            for (r, c, v), y in pairs:
                args += [r, c, v, y]
            return _spmm_acc(n_out, alpha, counts)(*args)

        o0 = run(50000, 0.5, [(ops["L0"], y_n2n), (ops["D1invB1"], y_e2n)])
        o1 = run(100000, 1.0 / 3.0, [(ops["L1"], y_e2e),
                                     (ops["D2B1TD1inv"], y_n2e),
                                     (ops["B2D3"], y_t2e)])
        o2 = run(50000, 0.5, [(ops["L2"], y_t2t), (ops["B2TD2inv"], y_e2t)])
        return o0, o1, o2

    x0, x1, x2 = ff(X0), ff(X1), ff(X2)
    x0, x1, x2 = layer(x0, x1, x2, W1, b1)
    x0, x1, x2 = layer(x0, x1, x2, W2, b2)
    x0, x1, x2 = layer(x0, x1, x2, W3, b3)

    s0, c0 = _pool(50000)(x0, ii(batch0))
    s1, c1 = _pool(100000)(x1, ii(batch1))
    s2, c2 = _pool(50000)(x2, ii(batch2))

    return _final(s0, c0, s1, c1, s2, c2, ff(Wout), ff(bout))


# speedup vs baseline: 7.2031x; 2.9814x over previous
import jax
import jax.numpy as jnp
from jax import lax
from jax.experimental import pallas as pl
from jax.experimental.pallas import tpu as pltpu
from jax.experimental.pallas import tpu_sc as plsc

F32 = jnp.float32
I32 = jnp.int32

NC = 2
NS = 16
C = 128
NB = 4
EPAD = C * NS * NB
WC = 200

_mesh = plsc.VectorSubcoreMesh(core_axis_name="c", subcore_axis_name="s")
_sc_params = pltpu.CompilerParams(
    needs_layout_passes=False, use_tc_tiling_on_sc=False)


def _cdiv(a, b):
    return -(-a // b)


def _e16(e):
    return jnp.full((16,), e, I32)


def _spmm_acc(n_out, alpha, edge_counts):
    ACC = _cdiv(n_out, C) * C
    nops = len(edge_counts)

    def body(*refs):
        ins = refs[: 4 * nops]
        out_h = refs[4 * nops]
        sc = refs[4 * nops + 1:]
        idxv = sc[0:NB]
        colv = sc[NB:2 * NB]
        valv = sc[2 * NB:3 * NB]
        gbuf = sc[3 * NB:4 * NB]
        obuf = sc[4 * NB]
        acc = sc[4 * NB + 1]
        sem_m = sc[4 * NB + 2:4 * NB + 2 + NB]
        sem_g = sc[4 * NB + 2 + NB:4 * NB + 2 + 2 * NB]
        sem_s = sc[4 * NB + 2 + 2 * NB:4 * NB + 2 + 3 * NB]

        cid = lax.axis_index("c")
        sid = lax.axis_index("s")

        zero16 = jnp.zeros((16,), F32)

        @pl.loop(0, C)
        def _zg(r):
            gbuf[0][r, pl.ds(0, 16)] = zero16

        n_zc = ACC // C

        @pl.loop(0, _cdiv(n_zc, NS))
        def _za(j):
            z = sid + j * NS

            @pl.when(z < n_zc)
            def _():
                pltpu.sync_copy(gbuf[0], acc.at[pl.ds(z * C, C)])

        plsc.subcore_barrier()

        for oi in range(nops):
            rows_h, cols_h, vals_h, y2_h = ins[4 * oi: 4 * oi + 4]
            E = edge_counts[oi]
            n_ch = E // C
            M = n_ch // NS

            def issue_meta(k, ch, rows_h=rows_h, cols_h=cols_h, vals_h=vals_h):
                base = ch * C
                pltpu.async_copy(rows_h.at[pl.ds(base, C)], idxv[k], sem_m[k])
                pltpu.async_copy(cols_h.at[pl.ds(base, C)], colv[k], sem_m[k])
                pltpu.async_copy(vals_h.at[pl.ds(base, C)], valv[k], sem_m[k])

            def wait_meta(k, rows_h=rows_h, cols_h=cols_h, vals_h=vals_h):
                pltpu.make_async_copy(
                    rows_h.at[pl.ds(0, C)], idxv[k], sem_m[k]).wait()
                pltpu.make_async_copy(
                    cols_h.at[pl.ds(0, C)], colv[k], sem_m[k]).wait()
                pltpu.make_async_copy(
                    vals_h.at[pl.ds(0, C)], valv[k], sem_m[k]).wait()

            def xform_gather(k, y2_h=y2_h):
                @pl.loop(0, C // 16)
                def _(g):
                    cv = colv[k][pl.ds(g * 16, 16)]
                    colv[k][pl.ds(g * 16, 16)] = cv * 2 + cid

                pltpu.async_copy(y2_h.at[colv[k]], gbuf[k], sem_g[k])

            def wait_gather(k, y2_h=y2_h):
                pltpu.make_async_copy(
                    y2_h.at[pl.ds(0, C)], gbuf[k], sem_g[k]).wait()

            def scale(k):
                @pl.loop(0, C // 16)
                def _(g):
                    vv = valv[k][pl.ds(g * 16, 16)]
                    for e in range(16):
                        sv = vv.at[_e16(e)].get(mode="promise_in_bounds")
                        r = g * 16 + e
                        gbuf[k][r, pl.ds(0, 16)] = (
                            gbuf[k][r, pl.ds(0, 16)] * sv)

            def issue_scatter(k):
                pltpu.async_copy(gbuf[k], acc.at[idxv[k]], sem_s[k], add=True)

            def wait_scatter(k):
                pltpu.make_async_copy(
                    gbuf[k], acc.at[pl.ds(0, C)], sem_s[k]).wait()

            def chunk_of(j):
                return sid + j * NS

            issue_meta(0, chunk_of(0))
            issue_meta(1, chunk_of(1))
            wait_meta(0)
            xform_gather(0)

            @pl.loop(0, M, step=NB)
            def _main(j0):
                for t in range(NB):
                    k = t
                    j = j0 + t
                    wait_gather(k)

                    @pl.when(j >= 2)
                    def _():
                        wait_scatter((t + 2) % NB)

                    @pl.when(j + 2 < M)
                    def _():
                        issue_meta((t + 2) % NB, chunk_of(j + 2))

                    @pl.when(j + 1 < M)
                    def _():
                        wait_meta((t + 1) % NB)
                        xform_gather((t + 1) % NB)

                    scale(k)
                    issue_scatter(k)

            wait_scatter((M - 2) % NB)
            wait_scatter((M - 1) % NB)

        plsc.subcore_barrier()

        n_wc = n_out // WC
        a = jnp.float32(alpha)

        @pl.loop(0, _cdiv(n_wc, NS))
        def _wb(j):
            w = sid + j * NS

            @pl.when(w < n_wc)
            def _():
                pltpu.sync_copy(acc.at[pl.ds(w * WC, WC)], obuf)

                @pl.loop(0, WC)
                def _r(r):
                    lo = obuf[r, pl.ds(0, 16)]
                    obuf[r, pl.ds(0, 16)] = jnp.maximum(lo, 0.0) * a

                pltpu.sync_copy(obuf, out_h.at[cid, pl.ds(w * WC, WC)])

    return pl.kernel(
        body,
        out_type=jax.ShapeDtypeStruct((NC, n_out, 16), F32),
        mesh=_mesh,
        compiler_params=_sc_params,
        scratch_types=(
            [pltpu.VMEM((C,), I32) for _ in range(NB)]
            + [pltpu.VMEM((C,), I32) for _ in range(NB)]
            + [pltpu.VMEM((C,), F32) for _ in range(NB)]
            + [pltpu.VMEM((C, 16), F32) for _ in range(NB)]
            + [pltpu.VMEM((WC, 16), F32)]
            + [pltpu.VMEM_SHARED((ACC, 16), F32)]
            + [pltpu.SemaphoreType.DMA for _ in range(3 * NB)]
        ),
    )


def _pool(n_rows):
    ACC = 128

    def body(x_h, b_h, sums_h, cnts_h, idxv, gbuf, onesb, obuf, acc_s, acc_c):
        cid = lax.axis_index("c")
        sid = lax.axis_index("s")

        zero16 = jnp.zeros((16,), F32)
        one16 = jnp.ones((16,), F32)

        @pl.loop(0, C)
        def _init(r):
            gbuf[r, pl.ds(0, 16)] = zero16
            onesb[r, pl.ds(0, 16)] = one16

        @pl.when(sid == 0)
        def _():
            pltpu.sync_copy(gbuf, acc_s)
            pltpu.sync_copy(gbuf, acc_c)

        plsc.subcore_barrier()

        lane = lax.iota(I32, 16)
        n_ch = _cdiv(n_rows, C)

        @pl.loop(0, _cdiv(n_ch, NS))
        def _rows(j):
            ch = sid + j * NS

            @pl.when(ch < n_ch)
            def _():
                base0 = ch * C
                base = jnp.minimum(base0, n_rows - C)
                lane_lo = base0 - base
                pltpu.sync_copy(x_h.at[cid, pl.ds(base, C)], gbuf)
                pltpu.sync_copy(b_h.at[pl.ds(base, C)], idxv)
                for g in range(C // 16):
                    bv = idxv[pl.ds(g * 16, 16)]
                    ok = (bv >= 0) & (bv < 64) & ((lane + g * 16) >= lane_lo)
                    idxv[pl.ds(g * 16, 16)] = jnp.where(ok, bv, 64)
                pltpu.sync_copy(gbuf, acc_s.at[idxv], add=True)
                pltpu.sync_copy(onesb, acc_c.at[idxv], add=True)

        plsc.subcore_barrier()

        @pl.when(sid == 0)
        def _():
            pltpu.sync_copy(acc_s.at[pl.ds(0, 64)], obuf)
            pltpu.sync_copy(obuf, sums_h.at[cid])
            pltpu.sync_copy(acc_c.at[pl.ds(0, 64)], obuf)
            pltpu.sync_copy(obuf, cnts_h.at[cid])

    return pl.kernel(
        body,
        out_type=(jax.ShapeDtypeStruct((NC, 64, 16), F32),
                  jax.ShapeDtypeStruct((NC, 64, 16), F32)),
        mesh=_mesh,
        compiler_params=_sc_params,
        scratch_types=[
            pltpu.VMEM((C,), I32),
            pltpu.VMEM((C, 16), F32),
            pltpu.VMEM((C, 16), F32),
            pltpu.VMEM((64, 16), F32),
            pltpu.VMEM_SHARED((ACC, 16), F32),
            pltpu.VMEM_SHARED((ACC, 16), F32),
        ],
    )


def _dense(X, Ws, bs, block_rows=1000):
    split = X.ndim == 3
    n = X.shape[1] if split else X.shape[0]
    k = 32 if split else X.shape[1]
    ny = len(Ws)
    W = jnp.stack(Ws)
    b = jnp.stack(bs).reshape(ny, 1, 32)

    def body(x_ref, w_ref, b_ref, *outs):
        if split:
            x = jnp.concatenate([x_ref[0], x_ref[1]], axis=1)
        else:
            x = x_ref[...]
        for i in range(ny):
            outs[i][...] = (
                jnp.dot(x, w_ref[i], preferred_element_type=F32) + b_ref[i]
            )

    if split:
        x_spec = pl.BlockSpec((2, block_rows, 16), lambda i: (0, i, 0))
    else:
        x_spec = pl.BlockSpec((block_rows, k), lambda i: (i, 0))
    return pl.pallas_call(
        body,
        grid=(n // block_rows,),
        in_specs=[
            x_spec,
            pl.BlockSpec((ny, k, 32), lambda i: (0, 0, 0)),
            pl.BlockSpec((ny, 1, 32), lambda i: (0, 0, 0)),
        ],
        out_specs=[pl.BlockSpec((block_rows, 32), lambda i: (i, 0))] * ny,
        out_shape=[jax.ShapeDtypeStruct((n, 32), F32)] * ny,
    )(X, W, b)


def _final(s0, c0, s1, c1, s2, c2, Wout, bout):
    def body(s0r, c0r, s1r, c1r, s2r, c2r, wr, br, outr):
        def pooled(sr, cr):
            return jnp.concatenate(
                [sr[0] / jnp.maximum(cr[0], 1.0),
                 sr[1] / jnp.maximum(cr[1], 1.0)], axis=1)

        cat = jnp.concatenate(
            [pooled(s0r, c0r), pooled(s1r, c1r), pooled(s2r, c2r)], axis=1)
        z = jnp.dot(cat, wr[...], preferred_element_type=F32) + br[...]
        z = z - jnp.max(z, axis=1, keepdims=True)
        ez = jnp.exp(z)
        outr[...] = ez / jnp.sum(ez, axis=1, keepdims=True)

    return pl.pallas_call(
        body,
        out_shape=jax.ShapeDtypeStruct((64, 32), F32),
    )(s0, c0, s1, c1, s2, c2, Wout, bout.reshape(1, 32))


def _pad_w(W, b):
    kin, kout = W.shape[1], W.shape[2]
    kin_pad = 128 if kin == 128 else 32
    Wp = jnp.zeros((7, kin_pad, 32), F32).at[:, :kin, :kout].set(W)
    bp = jnp.zeros((7, 32), F32).at[:, :kout].set(b)
    return Wp, bp


def _pad_edges(rows, cols, vals):
    e = rows.shape[0]
    ep = _cdiv(e, EPAD) * EPAD
    pad = (0, ep - e)
    return (jnp.pad(rows.astype(I32), pad),
            jnp.pad(cols.astype(I32), pad),
            jnp.pad(vals.astype(F32), pad))


def kernel(X0, X1, X2,
           L0_rows, L0_cols, L0_vals,
           L1_rows, L1_cols, L1_vals,
           L2_rows, L2_cols, L2_vals,
           B2D3_rows, B2D3_cols, B2D3_vals,
           D2B1TD1inv_rows, D2B1TD1inv_cols, D2B1TD1inv_vals,
           D1invB1_rows, D1invB1_cols, D1invB1_vals,
           B2TD2inv_rows, B2TD2inv_cols, B2TD2inv_vals,
           batch0, batch1, batch2,
           W1, b1, W2, b2, W3, b3, Wout, bout):
    ii = lambda x: x.astype(I32)
    ff = lambda x: x.astype(F32)

    ops = {
        "L0": _pad_edges(L0_rows, L0_cols, L0_vals),
        "L1": _pad_edges(L1_rows, L1_cols, L1_vals),
        "L2": _pad_edges(L2_rows, L2_cols, L2_vals),
        "B2D3": _pad_edges(B2D3_rows, B2D3_cols, B2D3_vals),
        "D2B1TD1inv": _pad_edges(D2B1TD1inv_rows, D2B1TD1inv_cols,
                                 D2B1TD1inv_vals),
        "D1invB1": _pad_edges(D1invB1_rows, D1invB1_cols, D1invB1_vals),
        "B2TD2inv": _pad_edges(B2TD2inv_rows, B2TD2inv_cols, B2TD2inv_vals),
    }

    def layer(x0, x1, x2, W, b):
        Wp, bp = _pad_w(W, b)
        y_n2n, y_n2e = _dense(x0, [Wp[0], Wp[1]], [bp[0], bp[1]])
        y_e2e, y_e2n, y_e2t = _dense(x1, [Wp[2], Wp[3], Wp[4]],
                                     [bp[2], bp[3], bp[4]])
        y_t2e, y_t2t = _dense(x2, [Wp[5], Wp[6]], [bp[5], bp[6]])

        def run(n_out, alpha, pairs):
            counts = tuple(p[0][0].shape[0] for p in pairs)
            args = []
            for (r, c, v), y in pairs:
                args += [r, c, v, y.reshape(2 * y.shape[0], 16)]
            return _spmm_acc(n_out, alpha, counts)(*args)

        o0 = run(50000, 0.5, [(ops["L0"], y_n2n), (ops["D1invB1"], y_e2n)])
        o1 = run(100000, 1.0 / 3.0, [(ops["L1"], y_e2e),
                                     (ops["D2B1TD1inv"], y_n2e),
                                     (ops["B2D3"], y_t2e)])
        o2 = run(50000, 0.5, [(ops["L2"], y_t2t), (ops["B2TD2inv"], y_e2t)])
        return o0, o1, o2

    x0, x1, x2 = ff(X0), ff(X1), ff(X2)
    x0, x1, x2 = layer(x0, x1, x2, W1, b1)
    x0, x1, x2 = layer(x0, x1, x2, W2, b2)
    x0, x1, x2 = layer(x0, x1, x2, W3, b3)

    s0, c0 = _pool(50000)(x0, ii(batch0))
    s1, c1 = _pool(100000)(x1, ii(batch1))
    s2, c2 = _pool(50000)(x2, ii(batch2))

    return _final(s0, c0, s1, c1, s2, c2, ff(Wout), ff(bout))


# speedup vs baseline: 7.8627x; 1.0916x over previous
import jax
import jax.numpy as jnp
from jax import lax
from jax.experimental import pallas as pl
from jax.experimental.pallas import tpu as pltpu
from jax.experimental.pallas import tpu_sc as plsc

F32 = jnp.float32
I32 = jnp.int32

NC = 2
NS = 16
C = 128
NB = 4
CSET = 2 * C
EPAD = CSET * NS * NB
WC = 200

_mesh = plsc.VectorSubcoreMesh(core_axis_name="c", subcore_axis_name="s")
_sc_params = pltpu.CompilerParams(
    needs_layout_passes=False, use_tc_tiling_on_sc=False)


def _cdiv(a, b):
    return -(-a // b)


def _e16(e):
    return jnp.full((16,), e, I32)


def _spmm_acc(n_out, alpha, set_counts):
    ACC = _cdiv(n_out, C) * C
    nops = len(set_counts)

    def body(*refs):
        ins = refs[: 2 * nops]
        out_h = refs[2 * nops]
        sc = refs[2 * nops + 1:]
        mbuf = sc[0:NB]
        gbuf = sc[NB:2 * NB]
        obuf = sc[2 * NB]
        acc = sc[2 * NB + 1]
        sem_m = sc[2 * NB + 2:2 * NB + 2 + NB]
        sem_g = sc[2 * NB + 2 + NB:2 * NB + 2 + 2 * NB]
        sem_s = sc[2 * NB + 2 + 2 * NB:2 * NB + 2 + 3 * NB]

        cid = lax.axis_index("c")
        sid = lax.axis_index("s")

        zero16 = jnp.zeros((16,), F32)

        @pl.loop(0, C)
        def _zg(r):
            gbuf[0][r, pl.ds(0, 16)] = zero16

        n_zc = ACC // C

        @pl.loop(0, _cdiv(n_zc, NS))
        def _za(j):
            z = sid + j * NS

            @pl.when(z < n_zc)
            def _():
                pltpu.sync_copy(gbuf[0].at[pl.ds(0, C)],
                                acc.at[pl.ds(z * C, C)])

        plsc.subcore_barrier()

        for oi in range(nops):
            meta_h, y2_h = ins[2 * oi: 2 * oi + 2]
            n_set = set_counts[oi]
            M = n_set // NS

            def issue_meta(k, st, meta_h=meta_h):
                pltpu.async_copy(meta_h.at[st], mbuf[k], sem_m[k])

            def wait_meta(k, meta_h=meta_h):
                pltpu.make_async_copy(meta_h.at[0], mbuf[k], sem_m[k]).wait()

            def xform_gather(k, y2_h=y2_h):
                for h in range(2):
                    @pl.loop(0, C // 16)
                    def _(g, h=h):
                        cv = mbuf[k][2 + h, pl.ds(g * 16, 16)]
                        mbuf[k][2 + h, pl.ds(g * 16, 16)] = cv + cid

                for h in range(2):
                    pltpu.async_copy(
                        y2_h.at[mbuf[k].at[2 + h]],
                        gbuf[k].at[pl.ds(h * C, C)], sem_g[k])

            def wait_gather(k, y2_h=y2_h):
                for h in range(2):
                    pltpu.make_async_copy(
                        y2_h.at[pl.ds(0, C)],
                        gbuf[k].at[pl.ds(h * C, C)], sem_g[k]).wait()

            def scale(k):
                for h in range(2):
                    @pl.loop(0, C // 16)
                    def _(g, h=h):
                        vv = lax.bitcast_convert_type(
                            mbuf[k][4 + h, pl.ds(g * 16, 16)], F32)
                        for e in range(16):
                            sv = vv.at[_e16(e)].get(mode="promise_in_bounds")
                            r = h * C + g * 16 + e
                            gbuf[k][r, pl.ds(0, 16)] = (
                                gbuf[k][r, pl.ds(0, 16)] * sv)

            def issue_scatter(k):
                for h in range(2):
                    pltpu.async_copy(
                        gbuf[k].at[pl.ds(h * C, C)],
                        acc.at[mbuf[k].at[h]], sem_s[k], add=True)

            def wait_scatter(k):
                for h in range(2):
                    pltpu.make_async_copy(
                        gbuf[k].at[pl.ds(h * C, C)],
                        acc.at[pl.ds(0, C)], sem_s[k]).wait()

            def set_of(j):
                return sid + j * NS

            issue_meta(0, set_of(0))
            issue_meta(1, set_of(1))
            wait_meta(0)
            xform_gather(0)

            @pl.loop(0, M, step=NB)
            def _main(j0):
                for t in range(NB):
                    k = t
                    j = j0 + t
                    wait_gather(k)

                    @pl.when(j >= 2)
                    def _():
                        wait_scatter((t + 2) % NB)

                    @pl.when(j + 2 < M)
                    def _():
                        issue_meta((t + 2) % NB, set_of(j + 2))

                    @pl.when(j + 1 < M)
                    def _():
                        wait_meta((t + 1) % NB)
                        xform_gather((t + 1) % NB)

                    scale(k)
                    issue_scatter(k)

            wait_scatter((M - 2) % NB)
            wait_scatter((M - 1) % NB)

        plsc.subcore_barrier()

        n_wc = n_out // WC
        a = jnp.float32(alpha)

        @pl.loop(0, _cdiv(n_wc, NS))
        def _wb(j):
            w = sid + j * NS

            @pl.when(w < n_wc)
            def _():
                pltpu.sync_copy(acc.at[pl.ds(w * WC, WC)], obuf)

                @pl.loop(0, WC)
                def _r(r):
                    lo = obuf[r, pl.ds(0, 16)]
                    obuf[r, pl.ds(0, 16)] = jnp.maximum(lo, 0.0) * a

                pltpu.sync_copy(obuf, out_h.at[cid, pl.ds(w * WC, WC)])

    return pl.kernel(
        body,
        out_type=jax.ShapeDtypeStruct((NC, n_out, 16), F32),
        mesh=_mesh,
        compiler_params=_sc_params,
        scratch_types=(
            [pltpu.VMEM((6, C), I32) for _ in range(NB)]
            + [pltpu.VMEM((CSET, 16), F32) for _ in range(NB)]
            + [pltpu.VMEM((WC, 16), F32)]
            + [pltpu.VMEM_SHARED((ACC, 16), F32)]
            + [pltpu.SemaphoreType.DMA for _ in range(3 * NB)]
        ),
    )


def _pool(n_rows):
    ACC = 128

    def body(x_h, b_h, sums_h, cnts_h, idxv, gbuf, onesb, obuf, acc_s, acc_c):
        cid = lax.axis_index("c")
        sid = lax.axis_index("s")

        zero16 = jnp.zeros((16,), F32)
        one16 = jnp.ones((16,), F32)

        @pl.loop(0, C)
        def _init(r):
            gbuf[r, pl.ds(0, 16)] = zero16
            onesb[r, pl.ds(0, 16)] = one16

        @pl.when(sid == 0)
        def _():
            pltpu.sync_copy(gbuf, acc_s)
            pltpu.sync_copy(gbuf, acc_c)

        plsc.subcore_barrier()

        lane = lax.iota(I32, 16)
        n_ch = _cdiv(n_rows, C)

        @pl.loop(0, _cdiv(n_ch, NS))
        def _rows(j):
            ch = sid + j * NS

            @pl.when(ch < n_ch)
            def _():
                base0 = ch * C
                base = jnp.minimum(base0, n_rows - C)
                lane_lo = base0 - base
                pltpu.sync_copy(x_h.at[cid, pl.ds(base, C)], gbuf)
                pltpu.sync_copy(b_h.at[pl.ds(base, C)], idxv)
                for g in range(C // 16):
                    bv = idxv[pl.ds(g * 16, 16)]
                    ok = (bv >= 0) & (bv < 64) & ((lane + g * 16) >= lane_lo)
                    idxv[pl.ds(g * 16, 16)] = jnp.where(ok, bv, 64)
                pltpu.sync_copy(gbuf, acc_s.at[idxv], add=True)
                pltpu.sync_copy(onesb, acc_c.at[idxv], add=True)

        plsc.subcore_barrier()

        @pl.when(sid == 0)
        def _():
            pltpu.sync_copy(acc_s.at[pl.ds(0, 64)], obuf)
            pltpu.sync_copy(obuf, sums_h.at[cid])
            pltpu.sync_copy(acc_c.at[pl.ds(0, 64)], obuf)
            pltpu.sync_copy(obuf, cnts_h.at[cid])

    return pl.kernel(
        body,
        out_type=(jax.ShapeDtypeStruct((NC, 64, 16), F32),
                  jax.ShapeDtypeStruct((NC, 64, 16), F32)),
        mesh=_mesh,
        compiler_params=_sc_params,
        scratch_types=[
            pltpu.VMEM((C,), I32),
            pltpu.VMEM((C, 16), F32),
            pltpu.VMEM((C, 16), F32),
            pltpu.VMEM((64, 16), F32),
            pltpu.VMEM_SHARED((ACC, 16), F32),
            pltpu.VMEM_SHARED((ACC, 16), F32),
        ],
    )


def _dense(X, Ws, bs, block_rows=1000):
    split = X.ndim == 3
    n = X.shape[1] if split else X.shape[0]
    k = 32 if split else X.shape[1]
    ny = len(Ws)
    W = jnp.stack(Ws)
    b = jnp.stack(bs).reshape(ny, 1, 32)

    def body(x_ref, w_ref, b_ref, *outs):
        if split:
            x = jnp.concatenate([x_ref[0], x_ref[1]], axis=1)
        else:
            x = x_ref[...]
        for i in range(ny):
            outs[i][...] = (
                jnp.dot(x, w_ref[i], preferred_element_type=F32) + b_ref[i]
            )

    if split:
        x_spec = pl.BlockSpec((2, block_rows, 16), lambda i: (0, i, 0))
    else:
        x_spec = pl.BlockSpec((block_rows, k), lambda i: (i, 0))
    return pl.pallas_call(
        body,
        grid=(n // block_rows,),
        in_specs=[
            x_spec,
            pl.BlockSpec((ny, k, 32), lambda i: (0, 0, 0)),
            pl.BlockSpec((ny, 1, 32), lambda i: (0, 0, 0)),
        ],
        out_specs=[pl.BlockSpec((block_rows, 32), lambda i: (i, 0))] * ny,
        out_shape=[jax.ShapeDtypeStruct((n, 32), F32)] * ny,
    )(X, W, b)


def _final(s0, c0, s1, c1, s2, c2, Wout, bout):
    def body(s0r, c0r, s1r, c1r, s2r, c2r, wr, br, outr):
        def pooled(sr, cr):
            return jnp.concatenate(
                [sr[0] / jnp.maximum(cr[0], 1.0),
                 sr[1] / jnp.maximum(cr[1], 1.0)], axis=1)

        cat = jnp.concatenate(
            [pooled(s0r, c0r), pooled(s1r, c1r), pooled(s2r, c2r)], axis=1)
        z = jnp.dot(cat, wr[...], preferred_element_type=F32) + br[...]
        z = z - jnp.max(z, axis=1, keepdims=True)
        ez = jnp.exp(z)
        outr[...] = ez / jnp.sum(ez, axis=1, keepdims=True)

    return pl.pallas_call(
        body,
        out_shape=jax.ShapeDtypeStruct((64, 32), F32),
    )(s0, c0, s1, c1, s2, c2, Wout, bout.reshape(1, 32))


def _pad_w(W, b):
    kin, kout = W.shape[1], W.shape[2]
    kin_pad = 128 if kin == 128 else 32
    Wp = jnp.zeros((7, kin_pad, 32), F32).at[:, :kin, :kout].set(W)
    bp = jnp.zeros((7, 32), F32).at[:, :kout].set(b)
    return Wp, bp


def _prep_op(rows, cols, vals):
    e = rows.shape[0]
    ep = _cdiv(e, EPAD) * EPAD
    pad = (0, ep - e)
    r = jnp.pad(rows.astype(I32), pad)
    c = jnp.pad(cols.astype(I32), pad) * 2
    v = lax.bitcast_convert_type(jnp.pad(vals.astype(F32), pad), I32)
    ns = ep // CSET
    return jnp.concatenate(
        [r.reshape(ns, 2, C), c.reshape(ns, 2, C), v.reshape(ns, 2, C)],
        axis=1)


def kernel(X0, X1, X2,
           L0_rows, L0_cols, L0_vals,
           L1_rows, L1_cols, L1_vals,
           L2_rows, L2_cols, L2_vals,
           B2D3_rows, B2D3_cols, B2D3_vals,
           D2B1TD1inv_rows, D2B1TD1inv_cols, D2B1TD1inv_vals,
           D1invB1_rows, D1invB1_cols, D1invB1_vals,
           B2TD2inv_rows, B2TD2inv_cols, B2TD2inv_vals,
           batch0, batch1, batch2,
           W1, b1, W2, b2, W3, b3, Wout, bout):
    ii = lambda x: x.astype(I32)
    ff = lambda x: x.astype(F32)

    ops = {
        "L0": _prep_op(L0_rows, L0_cols, L0_vals),
        "L1": _prep_op(L1_rows, L1_cols, L1_vals),
        "L2": _prep_op(L2_rows, L2_cols, L2_vals),
        "B2D3": _prep_op(B2D3_rows, B2D3_cols, B2D3_vals),
        "D2B1TD1inv": _prep_op(D2B1TD1inv_rows, D2B1TD1inv_cols,
                               D2B1TD1inv_vals),
        "D1invB1": _prep_op(D1invB1_rows, D1invB1_cols, D1invB1_vals),
        "B2TD2inv": _prep_op(B2TD2inv_rows, B2TD2inv_cols, B2TD2inv_vals),
    }

    def layer(x0, x1, x2, W, b):
        Wp, bp = _pad_w(W, b)
        y_n2n, y_n2e = _dense(x0, [Wp[0], Wp[1]], [bp[0], bp[1]])
        y_e2e, y_e2n, y_e2t = _dense(x1, [Wp[2], Wp[3], Wp[4]],
                                     [bp[2], bp[3], bp[4]])
        y_t2e, y_t2t = _dense(x2, [Wp[5], Wp[6]], [bp[5], bp[6]])

        def run(n_out, alpha, pairs):
            counts = tuple(m.shape[0] for m, _ in pairs)
            args = []
            for m, y in pairs:
                args += [m, y.reshape(2 * y.shape[0], 16)]
            return _spmm_acc(n_out, alpha, counts)(*args)

        o0 = run(50000, 0.5, [(ops["L0"], y_n2n), (ops["D1invB1"], y_e2n)])
        o1 = run(100000, 1.0 / 3.0, [(ops["L1"], y_e2e),
                                     (ops["D2B1TD1inv"], y_n2e),
                                     (ops["B2D3"], y_t2e)])
        o2 = run(50000, 0.5, [(ops["L2"], y_t2t), (ops["B2TD2inv"], y_e2t)])
        return o0, o1, o2

    x0, x1, x2 = ff(X0), ff(X1), ff(X2)
    x0, x1, x2 = layer(x0, x1, x2, W1, b1)
    x0, x1, x2 = layer(x0, x1, x2, W2, b2)
    x0, x1, x2 = layer(x0, x1, x2, W3, b3)

    s0, c0 = _pool(50000)(x0, ii(batch0))
    s1, c1 = _pool(100000)(x1, ii(batch1))
    s2, c2 = _pool(50000)(x2, ii(batch2))

    return _final(s0, c0, s1, c1, s2, c2, ff(Wout), ff(bout))
